# baseline (speedup 1.0000x reference)
import functools
import jax
import jax.numpy as jnp
from jax import lax
from jax.experimental import pallas as pl
from jax.experimental.pallas import tpu as pltpu
from jax.experimental.pallas import tpu_sc as plsc

VOCAB = 1000000
B = 16384
K = 20
D = 64
W = D // 2
NW = 32
BW = B // NW
C = 64
NSTEP = BW // C
IDXROW = 128
XROWS_PER_CHUNK = (C * K) // IDXROW
TBLK = 8192


def _pack_words(x):
    u = lax.bitcast_convert_type(x, jnp.uint32) + 0x8000
    lo = u[:W] >> 16
    hi = u[W:] & jnp.uint32(0xFFFF0000)
    return lax.bitcast_convert_type(lo | hi, jnp.float32)


def _pack_kernel(ct_ref, xt_ref, o_ref):
    h = TBLK // 2
    cw = _pack_words(ct_ref[...])
    xw = _pack_words(xt_ref[...])
    z = jnp.concatenate([cw[:, :h], xw[:, :h], cw[:, h:], xw[:, h:]], axis=0)
    o_ref[...] = z.T


def _sc_kernel(cl_hbm, xl_hbm, tab_hbm, out_hbm,
               cidx, xidx, crow, xrow, obuf, sbuf, sem_a, sem_b):
    nc = 2
    wid = lax.axis_index("s") * nc + lax.axis_index("c")
    ck = C * K

    iota = lax.iota(jnp.int32, 16)
    hi_mask = iota < 4

    def unpk(v):
        return plsc.unpack(plsc.bitcast(v, jnp.bfloat16),
                           format=plsc.PackFormat.INTERLEAVED)

    pltpu.sync_copy(cl_hbm.at[wid], cidx)
    pltpu.sync_copy(xl_hbm.at[wid], xidx)

    def fire(s, boff, sem):
        pltpu.async_copy(tab_hbm.at[cidx.at[pl.ds(s * C, C)]],
                         crow.at[pl.ds(boff * C, C)], sem)
        for j in range(XROWS_PER_CHUNK):
            pltpu.async_copy(tab_hbm.at[xidx.at[s * XROWS_PER_CHUNK + j]],
                             xrow.at[pl.ds(boff * ck + j * IDXROW, IDXROW)],
                             sem)

    def wait_chunk(boff, sem):
        pltpu.make_async_copy(tab_hbm.at[pl.ds(0, C)],
                              crow.at[pl.ds(boff * C, C)], sem).wait()
        for j in range(XROWS_PER_CHUNK):
            pltpu.make_async_copy(
                tab_hbm.at[pl.ds(0, IDXROW)],
                xrow.at[pl.ds(boff * ck + j * IDXROW, IDXROW)], sem).wait()

    def compute(s, boff):
        def one_center(b, soff):
            ca1, cb1 = unpk(crow[boff * C + b, pl.ds(0, 16)])
            ca2, cb2 = unpk(crow[boff * C + b, pl.ds(16, 16)])
            for k in range(K):
                r = boff * ck + b * K + k
                a1, b1 = unpk(xrow[r, pl.ds(0, 16)])
                a2, b2 = unpk(xrow[r, pl.ds(16, 16)])
                p = ca1 * a1 + cb1 * b1 + (ca2 * a2 + cb2 * b2)
                plsc.store_scatter(sbuf, [soff + iota * 32 + k], p)
            s_lo = sbuf[pl.ds(soff, 16)]
            s_hi = sbuf[pl.ds(soff + 16, 16)]
            for l in range(1, 16):
                s_lo = s_lo + sbuf[pl.ds(soff + l * 32, 16)]
                s_hi = s_hi + sbuf[pl.ds(soff + l * 32 + 16, 16)]
            plsc.store_scatter(obuf, [b * K + iota], s_lo)
            plsc.store_scatter(obuf, [jnp.minimum(b * K + 16 + iota, ck - 1)],
                               s_hi, mask=hi_mask)

        def per_pair(t, _):
            one_center(2 * t, 0)
            one_center(2 * t + 1, 512)
            return 0

        lax.fori_loop(0, C // 2, per_pair, 0)
        pltpu.sync_copy(obuf, out_hbm.at[pl.ds(wid * BW * K + s * ck, ck)])

    fire(0, 0, sem_a)

    def dstep(t, _):
        s0 = 2 * t
        fire(s0 + 1, 1, sem_b)
        wait_chunk(0, sem_a)
        compute(s0, 0)

        @pl.when(s0 + 2 < NSTEP)
        def _():
            fire(s0 + 2, 0, sem_a)

        wait_chunk(1, sem_b)
        compute(s0 + 1, 1)
        return 0

    lax.fori_loop(0, NSTEP // 2, dstep, 0)


@jax.jit
def kernel(center_labels, context_labels, center_table, context_table):
    nblk = (VOCAB + TBLK - 1) // TBLK
    packed = pl.pallas_call(
        _pack_kernel,
        grid=(nblk,),
        in_specs=[
            pl.BlockSpec((D, TBLK), lambda i: (0, i)),
            pl.BlockSpec((D, TBLK), lambda i: (0, i)),
        ],
        out_specs=pl.BlockSpec((TBLK // 2, 4 * W), lambda i: (i, 0)),
        out_shape=jax.ShapeDtypeStruct((nblk * (TBLK // 2), 4 * W), jnp.float32),
        compiler_params=pltpu.CompilerParams(fuse_transposed_lhs_in_matmul=True),
    )(center_table.T, context_table.T)
    tab = packed.reshape(nblk * TBLK * 2, W)

    def remap(v, ctx):
        ib = v // TBLK
        l = v % TBLK
        half = (l >= TBLK // 2).astype(jnp.int32)
        ql = l - half * (TBLK // 2)
        return ((ib * (TBLK // 2) + ql) * 2 + half) * 2 + ctx

    mesh = plsc.VectorSubcoreMesh(core_axis_name="c", subcore_axis_name="s")
    k = functools.partial(
        pl.kernel,
        out_type=jax.ShapeDtypeStruct((B * K,), jnp.float32),
        mesh=mesh,
        compiler_params=pltpu.CompilerParams(needs_layout_passes=False,
                                             use_tc_tiling_on_sc=False),
        scratch_types=[
            pltpu.VMEM((BW,), jnp.int32),
            pltpu.VMEM((BW * K // IDXROW, IDXROW), jnp.int32),
            pltpu.VMEM((2 * C, W), jnp.float32),
            pltpu.VMEM((2 * C * K, W), jnp.float32),
            pltpu.VMEM((C * K,), jnp.float32),
            pltpu.VMEM((2 * 16 * 32,), jnp.float32),
            pltpu.SemaphoreType.DMA,
            pltpu.SemaphoreType.DMA,
        ],
    )(_sc_kernel)
    out = k(remap(center_labels, 0).reshape(NW, BW),
            remap(context_labels, 1).reshape(NW, BW * K // IDXROW, IDXROW),
            tab)
    return out.reshape(B, K)


# speedup vs baseline: 1.3350x; 1.3350x over previous
import functools
import jax
import jax.numpy as jnp
from jax import lax
from jax.experimental import pallas as pl
from jax.experimental.pallas import tpu as pltpu
from jax.experimental.pallas import tpu_sc as plsc

VOCAB = 1000000
B = 16384
K = 20
D = 64
W = D // 2
NW = 32
BW = B // NW
C = 32
NSTEP = BW // C
IDXROW = 128
XROWS_PER_CHUNK = (C * K) // IDXROW
TBLK = 8192


def _pack_words(x):
    u = lax.bitcast_convert_type(x, jnp.uint32) + 0x8000
    lo = u[:W] >> 16
    hi = u[W:] & jnp.uint32(0xFFFF0000)
    return lax.bitcast_convert_type(lo | hi, jnp.float32)


def _pack_kernel(ct_ref, xt_ref, o_ref):
    h = TBLK // 2
    cw = _pack_words(ct_ref[...])
    xw = _pack_words(xt_ref[...])
    z = jnp.concatenate([cw[:, :h], xw[:, :h], cw[:, h:], xw[:, h:]], axis=0)
    o_ref[...] = z.T


def _sc_kernel(cl_hbm, xl_hbm, tab_hbm, out_hbm,
               cidx, xidx, crow, xrow, obuf, sbuf, sem_a, sem_b):
    nc = 2
    wid = lax.axis_index("s") * nc + lax.axis_index("c")
    ck = C * K

    iota = lax.iota(jnp.int32, 16)
    hi_mask = iota < 4

    def unpk(v):
        return plsc.unpack(plsc.bitcast(v, jnp.bfloat16),
                           format=plsc.PackFormat.INTERLEAVED)

    pltpu.sync_copy(cl_hbm.at[wid], cidx)
    pltpu.sync_copy(xl_hbm.at[wid], xidx)

    def fire(s, boff, sem):
        pltpu.async_copy(tab_hbm.at[cidx.at[pl.ds(s * C, C)]],
                         crow.at[pl.ds(boff * C, C)], sem)
        for j in range(XROWS_PER_CHUNK):
            pltpu.async_copy(tab_hbm.at[xidx.at[s * XROWS_PER_CHUNK + j]],
                             xrow.at[pl.ds(boff * ck + j * IDXROW, IDXROW)],
                             sem)

    def wait_chunk(boff, sem):
        pltpu.make_async_copy(tab_hbm.at[pl.ds(0, C)],
                              crow.at[pl.ds(boff * C, C)], sem).wait()
        for j in range(XROWS_PER_CHUNK):
            pltpu.make_async_copy(
                tab_hbm.at[pl.ds(0, IDXROW)],
                xrow.at[pl.ds(boff * ck + j * IDXROW, IDXROW)], sem).wait()

    _dn = lax.GatherDimensionNumbers(offset_dims=(), collapsed_slice_dims=(0,),
                                     start_index_map=(0,))

    def take(p, idx):
        return lax.gather(p, idx[:, None], _dn, slice_sizes=(1,),
                          mode=lax.GatherScatterMode.PROMISE_IN_BOUNDS)
    perms = [iota ^ 8, iota ^ 4, iota ^ 2, iota ^ 1]
    kmasks = [iota == (kk % 16) for kk in range(K)]

    def lanesum(p):
        for pm in perms:
            p = p + take(p, pm)
        return p

    def compute(s, boff):
        def per_center(b, _):
            ca1, cb1 = unpk(crow[boff * C + b, pl.ds(0, 16)])
            ca2, cb2 = unpk(crow[boff * C + b, pl.ds(16, 16)])
            s_lo = s_hi = None
            for k in range(K):
                r = boff * ck + b * K + k
                a1, b1 = unpk(xrow[r, pl.ds(0, 16)])
                a2, b2 = unpk(xrow[r, pl.ds(16, 16)])
                t = lanesum(ca1 * a1 + cb1 * b1 + (ca2 * a2 + cb2 * b2))
                if k == 0:
                    s_lo = t
                elif k == 16:
                    s_hi = t
                elif k < 16:
                    s_lo = jnp.where(kmasks[k], t, s_lo)
                else:
                    s_hi = jnp.where(kmasks[k], t, s_hi)
            plsc.store_scatter(obuf, [b * K + iota], s_lo)
            plsc.store_scatter(obuf, [jnp.minimum(b * K + 16 + iota, ck - 1)],
                               s_hi, mask=hi_mask)
            return 0

        lax.fori_loop(0, C, per_center, 0)
        pltpu.sync_copy(obuf, out_hbm.at[pl.ds(wid * BW * K + s * ck, ck)])

    fire(0, 0, sem_a)

    def dstep(t, _):
        s0 = 2 * t
        fire(s0 + 1, 1, sem_b)
        wait_chunk(0, sem_a)
        compute(s0, 0)

        @pl.when(s0 + 2 < NSTEP)
        def _():
            fire(s0 + 2, 0, sem_a)

        wait_chunk(1, sem_b)
        compute(s0 + 1, 1)
        return 0

    lax.fori_loop(0, NSTEP // 2, dstep, 0)


@jax.jit
def kernel(center_labels, context_labels, center_table, context_table):
    nblk = (VOCAB + TBLK - 1) // TBLK
    packed = pl.pallas_call(
        _pack_kernel,
        grid=(nblk,),
        in_specs=[
            pl.BlockSpec((D, TBLK), lambda i: (0, i)),
            pl.BlockSpec((D, TBLK), lambda i: (0, i)),
        ],
        out_specs=pl.BlockSpec((TBLK // 2, 4 * W), lambda i: (i, 0)),
        out_shape=jax.ShapeDtypeStruct((nblk * (TBLK // 2), 4 * W), jnp.float32),
        compiler_params=pltpu.CompilerParams(fuse_transposed_lhs_in_matmul=True),
    )(center_table.T, context_table.T)
    tab = packed.reshape(nblk * TBLK * 2, W)

    def remap(v, ctx):
        ib = v // TBLK
        l = v % TBLK
        half = (l >= TBLK // 2).astype(jnp.int32)
        ql = l - half * (TBLK // 2)
        return ((ib * (TBLK // 2) + ql) * 2 + half) * 2 + ctx

    mesh = plsc.VectorSubcoreMesh(core_axis_name="c", subcore_axis_name="s")
    k = functools.partial(
        pl.kernel,
        out_type=jax.ShapeDtypeStruct((B * K,), jnp.float32),
        mesh=mesh,
        compiler_params=pltpu.CompilerParams(needs_layout_passes=False,
                                             use_tc_tiling_on_sc=False),
        scratch_types=[
            pltpu.VMEM((BW,), jnp.int32),
            pltpu.VMEM((BW * K // IDXROW, IDXROW), jnp.int32),
            pltpu.VMEM((2 * C, W), jnp.float32),
            pltpu.VMEM((2 * C * K, W), jnp.float32),
            pltpu.VMEM((C * K,), jnp.float32),
            pltpu.VMEM((2 * 16 * 32,), jnp.float32),
            pltpu.SemaphoreType.DMA,
            pltpu.SemaphoreType.DMA,
        ],
    )(_sc_kernel)
    out = k(remap(center_labels, 0).reshape(NW, BW),
            remap(context_labels, 1).reshape(NW, BW * K // IDXROW, IDXROW),
            tab)
    return out.reshape(B, K)


# speedup vs baseline: 1.3742x; 1.0293x over previous
import functools
import jax
import jax.numpy as jnp
from jax import lax
from jax.experimental import pallas as pl
from jax.experimental.pallas import tpu as pltpu
from jax.experimental.pallas import tpu_sc as plsc

VOCAB = 1000000
B = 16384
K = 20
D = 64
W = D // 2
NW = 32
BW = B // NW
C = 32
NSTEP = BW // C
IDXROW = 128
XROWS_PER_CHUNK = (C * K) // IDXROW
TBLK = 16384


def _pack_words(x):
    u = lax.bitcast_convert_type(x, jnp.uint32) + 0x8000
    lo = u[:W] >> 16
    hi = u[W:] & jnp.uint32(0xFFFF0000)
    return lax.bitcast_convert_type(lo | hi, jnp.float32)


def _pack_kernel(ct_ref, xt_ref, o_ref):
    h = TBLK // 2
    cw = _pack_words(ct_ref[...])
    xw = _pack_words(xt_ref[...])
    z = jnp.concatenate([cw[:, :h], xw[:, :h], cw[:, h:], xw[:, h:]], axis=0)
    o_ref[...] = z.T


def _sc_kernel(cl_hbm, xl_hbm, tab_hbm, out_hbm,
               cidx, xidx, crow, xrow, obuf, sbuf, sem_a, sem_b):
    nc = 2
    wid = lax.axis_index("s") * nc + lax.axis_index("c")
    ck = C * K

    iota = lax.iota(jnp.int32, 16)
    hi_mask = iota < 4

    def unpk(v):
        return plsc.unpack(plsc.bitcast(v, jnp.bfloat16),
                           format=plsc.PackFormat.INTERLEAVED)

    pltpu.sync_copy(cl_hbm.at[wid], cidx)
    pltpu.sync_copy(xl_hbm.at[wid], xidx)

    def fire(s, boff, sem):
        pltpu.async_copy(tab_hbm.at[cidx.at[pl.ds(s * C, C)]],
                         crow.at[pl.ds(boff * C, C)], sem)
        for j in range(XROWS_PER_CHUNK):
            pltpu.async_copy(tab_hbm.at[xidx.at[s * XROWS_PER_CHUNK + j]],
                             xrow.at[pl.ds(boff * ck + j * IDXROW, IDXROW)],
                             sem)

    def wait_chunk(boff, sem):
        pltpu.make_async_copy(tab_hbm.at[pl.ds(0, C)],
                              crow.at[pl.ds(boff * C, C)], sem).wait()
        for j in range(XROWS_PER_CHUNK):
            pltpu.make_async_copy(
                tab_hbm.at[pl.ds(0, IDXROW)],
                xrow.at[pl.ds(boff * ck + j * IDXROW, IDXROW)], sem).wait()

    _dn = lax.GatherDimensionNumbers(offset_dims=(), collapsed_slice_dims=(0,),
                                     start_index_map=(0,))

    def take(p, idx):
        return lax.gather(p, idx[:, None], _dn, slice_sizes=(1,),
                          mode=lax.GatherScatterMode.PROMISE_IN_BOUNDS)
    perms = [iota ^ 8, iota ^ 4, iota ^ 2, iota ^ 1]
    kmasks = [iota == (kk % 16) for kk in range(K)]

    def lanesum(p):
        for pm in perms:
            p = p + take(p, pm)
        return p

    def compute(s, boff):
        def per_center(b, _):
            ca1, cb1 = unpk(crow[boff * C + b, pl.ds(0, 16)])
            ca2, cb2 = unpk(crow[boff * C + b, pl.ds(16, 16)])
            s_lo = s_hi = None
            for k in range(K):
                r = boff * ck + b * K + k
                a1, b1 = unpk(xrow[r, pl.ds(0, 16)])
                a2, b2 = unpk(xrow[r, pl.ds(16, 16)])
                t = lanesum(ca1 * a1 + cb1 * b1 + (ca2 * a2 + cb2 * b2))
                if k == 0:
                    s_lo = t
                elif k == 16:
                    s_hi = t
                elif k < 16:
                    s_lo = jnp.where(kmasks[k], t, s_lo)
                else:
                    s_hi = jnp.where(kmasks[k], t, s_hi)
            plsc.store_scatter(obuf, [b * K + iota], s_lo)
            plsc.store_scatter(obuf, [jnp.minimum(b * K + 16 + iota, ck - 1)],
                               s_hi, mask=hi_mask)
            return 0

        lax.fori_loop(0, C, per_center, 0)
        pltpu.sync_copy(obuf, out_hbm.at[pl.ds(wid * BW * K + s * ck, ck)])

    fire(0, 0, sem_a)

    def dstep(t, _):
        s0 = 2 * t
        fire(s0 + 1, 1, sem_b)
        wait_chunk(0, sem_a)
        compute(s0, 0)

        @pl.when(s0 + 2 < NSTEP)
        def _():
            fire(s0 + 2, 0, sem_a)

        wait_chunk(1, sem_b)
        compute(s0 + 1, 1)
        return 0

    lax.fori_loop(0, NSTEP // 2, dstep, 0)


@jax.jit
def kernel(center_labels, context_labels, center_table, context_table):
    nblk = (VOCAB + TBLK - 1) // TBLK
    packed = pl.pallas_call(
        _pack_kernel,
        grid=(nblk,),
        in_specs=[
            pl.BlockSpec((D, TBLK), lambda i: (0, i)),
            pl.BlockSpec((D, TBLK), lambda i: (0, i)),
        ],
        out_specs=pl.BlockSpec((TBLK // 2, 4 * W), lambda i: (i, 0)),
        out_shape=jax.ShapeDtypeStruct((nblk * (TBLK // 2), 4 * W), jnp.float32),
        compiler_params=pltpu.CompilerParams(fuse_transposed_lhs_in_matmul=True),
    )(center_table.T, context_table.T)
    tab = packed.reshape(nblk * TBLK * 2, W)

    def remap(v, ctx):
        ib = v // TBLK
        l = v % TBLK
        half = (l >= TBLK // 2).astype(jnp.int32)
        ql = l - half * (TBLK // 2)
        return ((ib * (TBLK // 2) + ql) * 2 + half) * 2 + ctx

    mesh = plsc.VectorSubcoreMesh(core_axis_name="c", subcore_axis_name="s")
    k = functools.partial(
        pl.kernel,
        out_type=jax.ShapeDtypeStruct((B * K,), jnp.float32),
        mesh=mesh,
        compiler_params=pltpu.CompilerParams(needs_layout_passes=False,
                                             use_tc_tiling_on_sc=False),
        scratch_types=[
            pltpu.VMEM((BW,), jnp.int32),
            pltpu.VMEM((BW * K // IDXROW, IDXROW), jnp.int32),
            pltpu.VMEM((2 * C, W), jnp.float32),
            pltpu.VMEM((2 * C * K, W), jnp.float32),
            pltpu.VMEM((C * K,), jnp.float32),
            pltpu.VMEM((2 * 16 * 32,), jnp.float32),
            pltpu.SemaphoreType.DMA,
            pltpu.SemaphoreType.DMA,
        ],
    )(_sc_kernel)
    out = k(remap(center_labels, 0).reshape(NW, BW),
            remap(context_labels, 1).reshape(NW, BW * K // IDXROW, IDXROW),
            tab)
    return out.reshape(B, K)


# speedup vs baseline: 1.3814x; 1.0052x over previous
import functools
import jax
import jax.numpy as jnp
from jax import lax
from jax.experimental import pallas as pl
from jax.experimental.pallas import tpu as pltpu
from jax.experimental.pallas import tpu_sc as plsc

VOCAB = 1000000
B = 16384
K = 20
D = 64
W = D // 2
NW = 32
BW = B // NW
C = 32
NSTEP = BW // C
IDXROW = 128
XROWS_PER_CHUNK = (C * K) // IDXROW
TBLK = 32768


def _pack_words(x):
    u = lax.bitcast_convert_type(x, jnp.uint32) + 0x8000
    lo = u[:W] >> 16
    hi = u[W:] & jnp.uint32(0xFFFF0000)
    return lax.bitcast_convert_type(lo | hi, jnp.float32)


def _pack_kernel(ct_ref, xt_ref, o_ref):
    h = TBLK // 2
    cw = _pack_words(ct_ref[...])
    xw = _pack_words(xt_ref[...])
    z = jnp.concatenate([cw[:, :h], xw[:, :h], cw[:, h:], xw[:, h:]], axis=0)
    o_ref[...] = z.T


def _sc_kernel(cl_hbm, xl_hbm, tab_hbm, out_hbm,
               cidx, xidx, crow, xrow, obuf, sbuf, sem_a, sem_b):
    nc = 2
    wid = lax.axis_index("s") * nc + lax.axis_index("c")
    ck = C * K

    iota = lax.iota(jnp.int32, 16)
    hi_mask = iota < 4

    def unpk(v):
        return plsc.unpack(plsc.bitcast(v, jnp.bfloat16),
                           format=plsc.PackFormat.INTERLEAVED)

    pltpu.sync_copy(cl_hbm.at[wid], cidx)
    pltpu.sync_copy(xl_hbm.at[wid], xidx)

    def fire(s, boff, sem):
        pltpu.async_copy(tab_hbm.at[cidx.at[pl.ds(s * C, C)]],
                         crow.at[pl.ds(boff * C, C)], sem)
        for j in range(XROWS_PER_CHUNK):
            pltpu.async_copy(tab_hbm.at[xidx.at[s * XROWS_PER_CHUNK + j]],
                             xrow.at[pl.ds(boff * ck + j * IDXROW, IDXROW)],
                             sem)

    def wait_chunk(boff, sem):
        pltpu.make_async_copy(tab_hbm.at[pl.ds(0, C)],
                              crow.at[pl.ds(boff * C, C)], sem).wait()
        for j in range(XROWS_PER_CHUNK):
            pltpu.make_async_copy(
                tab_hbm.at[pl.ds(0, IDXROW)],
                xrow.at[pl.ds(boff * ck + j * IDXROW, IDXROW)], sem).wait()

    _dn = lax.GatherDimensionNumbers(offset_dims=(), collapsed_slice_dims=(0,),
                                     start_index_map=(0,))

    def take(p, idx):
        return lax.gather(p, idx[:, None], _dn, slice_sizes=(1,),
                          mode=lax.GatherScatterMode.PROMISE_IN_BOUNDS)
    perms = [iota ^ 8, iota ^ 4, iota ^ 2, iota ^ 1]
    kmasks = [iota == (kk % 16) for kk in range(K)]

    def lanesum(p):
        for pm in perms:
            p = p + take(p, pm)
        return p

    def compute(s, boff):
        def per_center(b, _):
            ca1, cb1 = unpk(crow[boff * C + b, pl.ds(0, 16)])
            ca2, cb2 = unpk(crow[boff * C + b, pl.ds(16, 16)])
            s_lo = s_hi = None
            for k in range(K):
                r = boff * ck + b * K + k
                a1, b1 = unpk(xrow[r, pl.ds(0, 16)])
                a2, b2 = unpk(xrow[r, pl.ds(16, 16)])
                t = lanesum(ca1 * a1 + cb1 * b1 + (ca2 * a2 + cb2 * b2))
                if k == 0:
                    s_lo = t
                elif k == 16:
                    s_hi = t
                elif k < 16:
                    s_lo = jnp.where(kmasks[k], t, s_lo)
                else:
                    s_hi = jnp.where(kmasks[k], t, s_hi)
            plsc.store_scatter(obuf, [b * K + iota], s_lo)
            plsc.store_scatter(obuf, [jnp.minimum(b * K + 16 + iota, ck - 1)],
                               s_hi, mask=hi_mask)
            return 0

        lax.fori_loop(0, C, per_center, 0)
        pltpu.sync_copy(obuf, out_hbm.at[pl.ds(wid * BW * K + s * ck, ck)])

    fire(0, 0, sem_a)

    def dstep(t, _):
        s0 = 2 * t
        fire(s0 + 1, 1, sem_b)
        wait_chunk(0, sem_a)
        compute(s0, 0)

        @pl.when(s0 + 2 < NSTEP)
        def _():
            fire(s0 + 2, 0, sem_a)

        wait_chunk(1, sem_b)
        compute(s0 + 1, 1)
        return 0

    lax.fori_loop(0, NSTEP // 2, dstep, 0)


@jax.jit
def kernel(center_labels, context_labels, center_table, context_table):
    nblk = (VOCAB + TBLK - 1) // TBLK
    packed = pl.pallas_call(
        _pack_kernel,
        grid=(nblk,),
        in_specs=[
            pl.BlockSpec((D, TBLK), lambda i: (0, i)),
            pl.BlockSpec((D, TBLK), lambda i: (0, i)),
        ],
        out_specs=pl.BlockSpec((TBLK // 2, 4 * W), lambda i: (i, 0)),
        out_shape=jax.ShapeDtypeStruct((nblk * (TBLK // 2), 4 * W), jnp.float32),
        compiler_params=pltpu.CompilerParams(fuse_transposed_lhs_in_matmul=True),
    )(center_table.T, context_table.T)
    tab = packed.reshape(nblk * TBLK * 2, W)

    def remap(v, ctx):
        ib = v // TBLK
        l = v % TBLK
        half = (l >= TBLK // 2).astype(jnp.int32)
        ql = l - half * (TBLK // 2)
        return ((ib * (TBLK // 2) + ql) * 2 + half) * 2 + ctx

    mesh = plsc.VectorSubcoreMesh(core_axis_name="c", subcore_axis_name="s")
    k = functools.partial(
        pl.kernel,
        out_type=jax.ShapeDtypeStruct((B * K,), jnp.float32),
        mesh=mesh,
        compiler_params=pltpu.CompilerParams(needs_layout_passes=False,
                                             use_tc_tiling_on_sc=False),
        scratch_types=[
            pltpu.VMEM((BW,), jnp.int32),
            pltpu.VMEM((BW * K // IDXROW, IDXROW), jnp.int32),
            pltpu.VMEM((2 * C, W), jnp.float32),
            pltpu.VMEM((2 * C * K, W), jnp.float32),
            pltpu.VMEM((C * K,), jnp.float32),
            pltpu.VMEM((2 * 16 * 32,), jnp.float32),
            pltpu.SemaphoreType.DMA,
            pltpu.SemaphoreType.DMA,
        ],
    )(_sc_kernel)
    out = k(remap(center_labels, 0).reshape(NW, BW),
            remap(context_labels, 1).reshape(NW, BW * K // IDXROW, IDXROW),
            tab)
    return out.reshape(B, K)


# speedup vs baseline: 1.3825x; 1.0009x over previous
import functools
import jax
import jax.numpy as jnp
from jax import lax
from jax.experimental import pallas as pl
from jax.experimental.pallas import tpu as pltpu
from jax.experimental.pallas import tpu_sc as plsc

VOCAB = 1000000
B = 16384
K = 20
D = 64
W = D // 2
NW = 32
BW = B // NW
C = 32
NSTEP = BW // C
IDXROW = 128
XROWS_PER_CHUNK = (C * K) // IDXROW
TBLK = 32768


def _pack_words(x):
    u = lax.bitcast_convert_type(x, jnp.uint32) + 0x8000
    lo = u[:W] >> 16
    hi = u[W:] & jnp.uint32(0xFFFF0000)
    return lax.bitcast_convert_type(lo | hi, jnp.float32)


def _pack_kernel(ct_ref, xt_ref, o_ref):
    h = TBLK // 2
    cw = _pack_words(ct_ref[...])
    xw = _pack_words(xt_ref[...])
    z = jnp.concatenate([cw[:, :h], xw[:, :h], cw[:, h:], xw[:, h:]], axis=0)
    o_ref[...] = z.T


def _sc_kernel(cl_hbm, xl_hbm, tab_hbm, out_hbm,
               cidx, xidx, crow, xrow, obuf, sem_a, sem_b):
    nc = 2
    wid = lax.axis_index("s") * nc + lax.axis_index("c")
    ck = C * K

    iota = lax.iota(jnp.int32, 16)
    hi_mask = iota < 4

    def unpk(v):
        return plsc.unpack(plsc.bitcast(v, jnp.bfloat16),
                           format=plsc.PackFormat.INTERLEAVED)

    pltpu.sync_copy(cl_hbm.at[wid], cidx)
    pltpu.sync_copy(xl_hbm.at[wid], xidx)

    def fire(s, boff, sem):
        pltpu.async_copy(tab_hbm.at[cidx.at[pl.ds(s * C, C)]],
                         crow.at[pl.ds(boff * C, C)], sem)
        for j in range(XROWS_PER_CHUNK):
            pltpu.async_copy(tab_hbm.at[xidx.at[s * XROWS_PER_CHUNK + j]],
                             xrow.at[pl.ds(boff * ck + j * IDXROW, IDXROW)],
                             sem)

    def wait_chunk(boff, sem):
        pltpu.make_async_copy(tab_hbm.at[pl.ds(0, C)],
                              crow.at[pl.ds(boff * C, C)], sem).wait()
        for j in range(XROWS_PER_CHUNK):
            pltpu.make_async_copy(
                tab_hbm.at[pl.ds(0, IDXROW)],
                xrow.at[pl.ds(boff * ck + j * IDXROW, IDXROW)], sem).wait()

    _dn = lax.GatherDimensionNumbers(offset_dims=(), collapsed_slice_dims=(0,),
                                     start_index_map=(0,))

    def take(p, idx):
        return lax.gather(p, idx[:, None], _dn, slice_sizes=(1,),
                          mode=lax.GatherScatterMode.PROMISE_IN_BOUNDS)
    perms = [iota ^ 8, iota ^ 4, iota ^ 2, iota ^ 1]
    kmasks = [iota == (kk % 16) for kk in range(K)]

    def lanesum(p):
        for pm in perms:
            p = p + take(p, pm)
        return p

    def compute(s, boff):
        def per_center(b, _):
            ca1, cb1 = unpk(crow[boff * C + b, pl.ds(0, 16)])
            ca2, cb2 = unpk(crow[boff * C + b, pl.ds(16, 16)])
            s_lo = s_hi = None
            for k in range(K):
                r = boff * ck + b * K + k
                a1, b1 = unpk(xrow[r, pl.ds(0, 16)])
                a2, b2 = unpk(xrow[r, pl.ds(16, 16)])
                t = lanesum(ca1 * a1 + cb1 * b1 + (ca2 * a2 + cb2 * b2))
                if k == 0:
                    s_lo = t
                elif k == 16:
                    s_hi = t
                elif k < 16:
                    s_lo = jnp.where(kmasks[k], t, s_lo)
                else:
                    s_hi = jnp.where(kmasks[k], t, s_hi)
            plsc.store_scatter(obuf, [b * K + iota], s_lo)
            plsc.store_scatter(obuf, [jnp.minimum(b * K + 16 + iota, ck - 1)],
                               s_hi, mask=hi_mask)
            return 0

        lax.fori_loop(0, C, per_center, 0)
        pltpu.sync_copy(obuf, out_hbm.at[pl.ds(wid * BW * K + s * ck, ck)])

    fire(0, 0, sem_a)

    def dstep(t, _):
        s0 = 2 * t
        fire(s0 + 1, 1, sem_b)
        wait_chunk(0, sem_a)
        compute(s0, 0)

        @pl.when(s0 + 2 < NSTEP)
        def _():
            fire(s0 + 2, 0, sem_a)

        wait_chunk(1, sem_b)
        compute(s0 + 1, 1)
        return 0

    lax.fori_loop(0, NSTEP // 2, dstep, 0)


@jax.jit
def kernel(center_labels, context_labels, center_table, context_table):
    nblk = (VOCAB + TBLK - 1) // TBLK
    packed = pl.pallas_call(
        _pack_kernel,
        grid=(nblk,),
        in_specs=[
            pl.BlockSpec((D, TBLK), lambda i: (0, i)),
            pl.BlockSpec((D, TBLK), lambda i: (0, i)),
        ],
        out_specs=pl.BlockSpec((TBLK // 2, 4 * W), lambda i: (i, 0)),
        out_shape=jax.ShapeDtypeStruct((nblk * (TBLK // 2), 4 * W), jnp.float32),
        compiler_params=pltpu.CompilerParams(fuse_transposed_lhs_in_matmul=True),
    )(center_table.T, context_table.T)
    tab = packed.reshape(nblk * TBLK * 2, W)

    def remap(v, ctx):
        ib = v // TBLK
        l = v % TBLK
        half = (l >= TBLK // 2).astype(jnp.int32)
        ql = l - half * (TBLK // 2)
        return ((ib * (TBLK // 2) + ql) * 2 + half) * 2 + ctx

    mesh = plsc.VectorSubcoreMesh(core_axis_name="c", subcore_axis_name="s")
    k = functools.partial(
        pl.kernel,
        out_type=jax.ShapeDtypeStruct((B * K,), jnp.float32),
        mesh=mesh,
        compiler_params=pltpu.CompilerParams(needs_layout_passes=False,
                                             use_tc_tiling_on_sc=False),
        scratch_types=[
            pltpu.VMEM((BW,), jnp.int32),
            pltpu.VMEM((BW * K // IDXROW, IDXROW), jnp.int32),
            pltpu.VMEM((2 * C, W), jnp.float32),
            pltpu.VMEM((2 * C * K, W), jnp.float32),
            pltpu.VMEM((C * K,), jnp.float32),
            pltpu.SemaphoreType.DMA,
            pltpu.SemaphoreType.DMA,
        ],
    )(_sc_kernel)
    out = k(remap(center_labels, 0).reshape(NW, BW),
            remap(context_labels, 1).reshape(NW, BW * K // IDXROW, IDXROW),
            tab)
    return out.reshape(B, K)
